# Initial kernel scaffold; baseline (speedup 1.0000x reference)
import jax
import jax.numpy as jnp
from jax.experimental import pallas as pl


def kernel(edge_index, f, ef, s, W1, b1, Wr1, br1, g1, be1, W2, b2, Wr2, br2, g2, be2, Wm1, bm1, gm1, bem1, Wm2, bm2, gm2, bem2, Wf, bf):
    raise NotImplementedError("write your pallas kernel here")


# speedup vs baseline: 6.0852x; 6.0852x over previous
import functools
import math

import jax
import jax.numpy as jnp
from jax import lax
from jax.experimental import pallas as pl
from jax.experimental.pallas import tpu as pltpu
from jax.experimental.pallas import tpu_sc as plsc

N = 10000
E = 320000
D = 128
S_DIM = 16
MLP_DIM = 128
NPRED = 1000
EPS = 1e-5
BN_S = 1.0 / math.sqrt(1.0 + EPS)

N_P = 10240
NBLK = N_P // 128
CH = 80
GRP = 16
EPT = CH * 128
E_P = 32 * EPT
ROWS2D = E_P // 128
RPT = N_P // 16
RC = RPT // 128

_MESH = plsc.VectorSubcoreMesh(core_axis_name="c", subcore_axis_name="s")


RPT_D = ROWS2D // 16
GRPD = 16
NGD = RPT_D // GRPD


@functools.partial(
    pl.kernel,
    out_type=jax.ShapeDtypeStruct((2, N_P, 128), jnp.float32),
    mesh=_MESH,
    scratch_types=[
        pltpu.VMEM((GRPD, 128), jnp.int32),
        pltpu.VMEM((128, 128), jnp.float32),
        pltpu.VMEM_SHARED((N_P, 128), jnp.float32),
    ],
)
def _degrees(idx_hbm, out_hbm, idxv, ones_v, acc_sh):
    c = lax.axis_index("c")
    s = lax.axis_index("s")
    r0 = s * RPT

    def fill0(i, carry):
        for k in range(8):
            ones_v[i, pl.ds(k * 16, 16)] = jnp.zeros((16,), jnp.float32)
        return carry

    lax.fori_loop(0, 128, fill0, 0)

    def zinit(j, carry):
        pltpu.sync_copy(ones_v, acc_sh.at[pl.ds(r0 + j * 128, 128)])
        return carry

    lax.fori_loop(0, RC, zinit, 0)

    def fill1(i, carry):
        for k in range(8):
            ones_v[i, pl.ds(k * 16, 16)] = jnp.float32(1.0) + jnp.zeros((16,), jnp.float32)
        return carry

    lax.fori_loop(0, 128, fill1, 0)
    plsc.subcore_barrier()

    def group(gi, carry):
        base = c * ROWS2D + s * RPT_D + gi * GRPD
        pltpu.sync_copy(idx_hbm.at[pl.ds(base, GRPD)], idxv)

        def step(j, carry2):
            pltpu.sync_copy(ones_v, acc_sh.at[idxv.at[j]], add=True)
            return carry2

        lax.fori_loop(0, GRPD, step, 0)
        return carry

    lax.fori_loop(0, NGD, group, 0)
    plsc.subcore_barrier()

    def wout(j, carry):
        sl = pl.ds(r0 + j * 128, 128)
        pltpu.sync_copy(acc_sh.at[sl], ones_v)
        pltpu.sync_copy(ones_v, out_hbm.at[c, sl])
        return carry

    lax.fori_loop(0, RC, wout, 0)


@functools.partial(
    pl.kernel,
    out_type=jax.ShapeDtypeStruct((2, N_P, D), jnp.float32),
    mesh=_MESH,
    scratch_types=[
        pltpu.VMEM((GRP, 128), jnp.int32),
        pltpu.VMEM((GRP, 128), jnp.int32),
        pltpu.VMEM((128, D), jnp.float32),
        pltpu.VMEM((128, D), jnp.float32),
        pltpu.VMEM_SHARED((N_P, D), jnp.float32),
        pltpu.SemaphoreType.DMA,
        pltpu.SemaphoreType.DMA,
    ],
)
def _msgpass(h_hbm, src_hbm, dst_hbm, out_hbm, srcv, dstv, rows0, rows1, agg_sh, sem0, sem1):
    c = lax.axis_index("c")
    s = lax.axis_index("s")
    t = c * 16 + s
    r0 = s * RPT

    def fill(i, carry):
        for k in range(8):
            rows0[i, pl.ds(k * 16, 16)] = jnp.zeros((16,), jnp.float32)
        return carry

    lax.fori_loop(0, 128, fill, 0)

    def zinit(j, carry):
        pltpu.sync_copy(rows0, agg_sh.at[pl.ds(r0 + j * 128, 128)])
        return carry

    lax.fori_loop(0, RC, zinit, 0)
    plsc.subcore_barrier()

    def group(gi, carry):
        base = t * CH + gi * GRP
        pltpu.sync_copy(src_hbm.at[pl.ds(base, GRP)], srcv)
        pltpu.sync_copy(dst_hbm.at[pl.ds(base, GRP)], dstv)

        def step(p, carry2):
            j0 = p * 2
            cp0 = pltpu.async_copy(h_hbm.at[srcv.at[j0]], rows0, sem0)
            cp1 = pltpu.async_copy(h_hbm.at[srcv.at[j0 + 1]], rows1, sem1)
            cp0.wait()
            pltpu.sync_copy(rows0, agg_sh.at[dstv.at[j0]], add=True)
            cp1.wait()
            pltpu.sync_copy(rows1, agg_sh.at[dstv.at[j0 + 1]], add=True)
            return carry2

        lax.fori_loop(0, GRP // 2, step, 0)
        return carry

    lax.fori_loop(0, CH // GRP, group, 0)
    plsc.subcore_barrier()

    def wout(j, carry):
        sl = pl.ds(r0 + j * 128, 128)
        pltpu.sync_copy(agg_sh.at[sl], rows0)
        pltpu.sync_copy(rows0, out_hbm.at[c, sl])
        return carry

    lax.fori_loop(0, RC, wout, 0)


def _pre1_body(cnt_ref, f_ref, o_ref):
    cout = cnt_ref[0][:, :1]
    dout = lax.rsqrt(jnp.maximum(cout, 1.0))
    o_ref[...] = f_ref[...] * dout


_CNT_SPEC = pl.BlockSpec((2, 128, 128), lambda i: (0, i, 0))
_ROW_SPEC = pl.BlockSpec((128, D), lambda i: (i, 0))
_AGG_SPEC = pl.BlockSpec((2, 128, D), lambda i: (0, i, 0))
_W_SPEC = pl.BlockSpec((D, D), lambda i: (0, 0))
_V_SPEC = pl.BlockSpec((1, D), lambda i: (0, 0))

_pre1 = pl.pallas_call(
    _pre1_body,
    grid=(NBLK,),
    in_specs=[_CNT_SPEC, _ROW_SPEC],
    out_specs=_ROW_SPEC,
    out_shape=jax.ShapeDtypeStruct((N_P, D), jnp.float32),
)


def _make_post(pool):
    def body(x_ref, aggp_ref, cnt_ref, W_ref, Wr_ref, b_ref, br_ref, g_ref, be_ref, *outs):
        i = pl.program_id(0)
        cnt = cnt_ref[...]
        cin = cnt[1][:, :1]
        din = lax.rsqrt(jnp.maximum(cin, 1.0))
        agg = (aggp_ref[0] + aggp_ref[1]) * din
        new = jnp.dot(agg, W_ref[...], preferred_element_type=jnp.float32) + b_ref[...]
        res = jnp.dot(x_ref[...], Wr_ref[...], preferred_element_type=jnp.float32) + br_ref[...]
        res = jnp.maximum(res, 0.0)
        h = (new + res) * BN_S * g_ref[...] + be_ref[...]
        if pool:
            (pool_ref,) = outs
            rows = i * 128 + lax.broadcasted_iota(jnp.int32, (128, 1), 0)
            ps = jnp.sum(jnp.where(rows < N, h, 0.0), axis=0, keepdims=True)

            @pl.when(i == 0)
            def _():
                pool_ref[...] = ps

            @pl.when(i > 0)
            def _():
                pool_ref[...] += ps
        else:
            h_ref, hs_ref = outs
            h_ref[...] = h
            cout = cnt[0][:, :1]
            dout = lax.rsqrt(jnp.maximum(cout, 1.0))
            hs_ref[...] = h * dout

    return body


_POST_IN_SPECS = [_ROW_SPEC, _AGG_SPEC, _CNT_SPEC,
                  _W_SPEC, _W_SPEC, _V_SPEC, _V_SPEC, _V_SPEC, _V_SPEC]

_post1 = pl.pallas_call(
    _make_post(False),
    grid=(NBLK,),
    in_specs=_POST_IN_SPECS,
    out_specs=[_ROW_SPEC, _ROW_SPEC],
    out_shape=[jax.ShapeDtypeStruct((N_P, D), jnp.float32),
               jax.ShapeDtypeStruct((N_P, D), jnp.float32)],
)

_post2 = pl.pallas_call(
    _make_post(True),
    grid=(NBLK,),
    in_specs=_POST_IN_SPECS,
    out_specs=pl.BlockSpec((1, D), lambda i: (0, 0)),
    out_shape=jax.ShapeDtypeStruct((1, D), jnp.float32),
)


def _head_body(pool_ref, s_ref, Wm1_ref, bm1_ref, gm1_ref, bem1_ref,
               Wm2_ref, bm2_ref, gm2_ref, bem2_ref, Wf_ref, bf_ref, o_ref):
    hg = jnp.maximum(pool_ref[...] * (1.0 / N), 0.0)
    x = jnp.dot(hg, Wm1_ref[0:D, :], preferred_element_type=jnp.float32)
    x = x + jnp.dot(s_ref[...], Wm1_ref[D:D + S_DIM, :], preferred_element_type=jnp.float32)
    x = jnp.maximum(x + bm1_ref[...], 0.0) * BN_S * gm1_ref[...] + bem1_ref[...]
    x = jnp.dot(x, Wm2_ref[...], preferred_element_type=jnp.float32) + bm2_ref[...]
    x = jnp.maximum(x, 0.0) * BN_S * gm2_ref[...] + bem2_ref[...]
    o = jnp.dot(x, Wf_ref[...], preferred_element_type=jnp.float32) + bf_ref[...]
    o_ref[...] = jnp.maximum(o, 0.0)


_head = pl.pallas_call(
    _head_body,
    out_shape=jax.ShapeDtypeStruct((1, NPRED), jnp.float32),
)


def kernel(edge_index, f, ef, s,
           W1, b1, Wr1, br1, g1, be1,
           W2, b2, Wr2, br2, g2, be2,
           Wm1, bm1, gm1, bem1,
           Wm2, bm2, gm2, bem2,
           Wf, bf):
    pad = (N + (jnp.arange(E_P - E, dtype=jnp.int32) % (N_P - N))).astype(jnp.int32)
    src_p = jnp.concatenate([edge_index[0], pad]).reshape(ROWS2D, 128)
    dst_p = jnp.concatenate([edge_index[1], pad]).reshape(ROWS2D, 128)
    f_p = jnp.pad(f, ((0, N_P - N), (0, 0)))

    cnt = _degrees(jnp.concatenate([src_p, dst_p], axis=0))
    h1s = _pre1(cnt, f_p)
    aggp1 = _msgpass(h1s, src_p, dst_p)
    h1, h2s = _post1(f_p, aggp1, cnt,
                     W1, Wr1, b1.reshape(1, D), br1.reshape(1, D),
                     g1.reshape(1, D), be1.reshape(1, D))
    aggp2 = _msgpass(h2s, src_p, dst_p)
    pool = _post2(h1, aggp2, cnt,
                  W2, Wr2, b2.reshape(1, D), br2.reshape(1, D),
                  g2.reshape(1, D), be2.reshape(1, D))
    out = _head(pool, s, Wm1, bm1.reshape(1, MLP_DIM), gm1.reshape(1, MLP_DIM),
                bem1.reshape(1, MLP_DIM), Wm2, bm2.reshape(1, MLP_DIM),
                gm2.reshape(1, MLP_DIM), bem2.reshape(1, MLP_DIM),
                Wf, bf.reshape(1, NPRED))
    return out


# speedup vs baseline: 7.1280x; 1.1714x over previous
import functools
import math

import jax
import jax.numpy as jnp
from jax import lax
from jax.experimental import pallas as pl
from jax.experimental.pallas import tpu as pltpu
from jax.experimental.pallas import tpu_sc as plsc

N = 10000
E = 320000
D = 128
S_DIM = 16
MLP_DIM = 128
NPRED = 1000
EPS = 1e-5
BN_S = 1.0 / math.sqrt(1.0 + EPS)

N_P = 10240
NBLK = N_P // 128
CH = 80
GRP = 16
EPT = CH * 128
E_P = 32 * EPT
ROWS2D = E_P // 128
RPT = N_P // 16
RC = RPT // 128

_MESH = plsc.VectorSubcoreMesh(core_axis_name="c", subcore_axis_name="s")


RPT_D = ROWS2D // 16
GRPD = 16
NGD = RPT_D // GRPD


@functools.partial(
    pl.kernel,
    out_type=jax.ShapeDtypeStruct((2, N_P, 128), jnp.float32),
    mesh=_MESH,
    scratch_types=[
        pltpu.VMEM((GRPD, 128), jnp.int32),
        pltpu.VMEM((128, 128), jnp.float32),
        pltpu.VMEM_SHARED((N_P, 128), jnp.float32),
    ],
)
def _degrees(idx_hbm, out_hbm, idxv, ones_v, acc_sh):
    c = lax.axis_index("c")
    s = lax.axis_index("s")
    r0 = s * RPT

    def fill0(i, carry):
        for k in range(8):
            ones_v[i, pl.ds(k * 16, 16)] = jnp.zeros((16,), jnp.float32)
        return carry

    lax.fori_loop(0, 128, fill0, 0)

    def zinit(j, carry):
        pltpu.sync_copy(ones_v, acc_sh.at[pl.ds(r0 + j * 128, 128)])
        return carry

    lax.fori_loop(0, RC, zinit, 0)

    def fill1(i, carry):
        for k in range(8):
            ones_v[i, pl.ds(k * 16, 16)] = jnp.float32(1.0) + jnp.zeros((16,), jnp.float32)
        return carry

    lax.fori_loop(0, 128, fill1, 0)
    plsc.subcore_barrier()

    def group(gi, carry):
        base = c * ROWS2D + s * RPT_D + gi * GRPD
        pltpu.sync_copy(idx_hbm.at[pl.ds(base, GRPD)], idxv)

        def step(j, carry2):
            pltpu.sync_copy(ones_v, acc_sh.at[idxv.at[j]], add=True)
            return carry2

        lax.fori_loop(0, GRPD, step, 0)
        return carry

    lax.fori_loop(0, NGD, group, 0)
    plsc.subcore_barrier()

    def wout(j, carry):
        sl = pl.ds(r0 + j * 128, 128)
        pltpu.sync_copy(acc_sh.at[sl], ones_v)
        pltpu.sync_copy(ones_v, out_hbm.at[c, sl])
        return carry

    lax.fori_loop(0, RC, wout, 0)


@functools.partial(
    pl.kernel,
    out_type=jax.ShapeDtypeStruct((2, N_P, D), jnp.float32),
    mesh=_MESH,
    scratch_types=[
        pltpu.VMEM((GRP, 128), jnp.int32),
        pltpu.VMEM((GRP, 128), jnp.int32),
        pltpu.VMEM((GRP, 128), jnp.int32),
        pltpu.VMEM((GRP, 128), jnp.int32),
        pltpu.VMEM((128, D), jnp.float32),
        pltpu.VMEM((128, D), jnp.float32),
        pltpu.VMEM_SHARED((N_P, D), jnp.float32),
        pltpu.SemaphoreType.DMA,
        pltpu.SemaphoreType.DMA,
        pltpu.SemaphoreType.DMA,
        pltpu.SemaphoreType.DMA,
    ],
)
def _msgpass(h_hbm, src_hbm, dst_hbm, out_hbm,
             srcv0, srcv1, dstv0, dstv1, rows0, rows1, agg_sh,
             semg0, semg1, sems0, sems1):
    c = lax.axis_index("c")
    s = lax.axis_index("s")
    t = c * 16 + s
    r0 = s * RPT

    def fill(i, carry):
        for k in range(8):
            rows0[i, pl.ds(k * 16, 16)] = jnp.zeros((16,), jnp.float32)
        return carry

    lax.fori_loop(0, 128, fill, 0)

    def zinit(j, carry):
        pltpu.sync_copy(rows0, agg_sh.at[pl.ds(r0 + j * 128, 128)])
        return carry

    lax.fori_loop(0, RC, zinit, 0)
    plsc.subcore_barrier()

    srcv = (srcv0, srcv1)
    dstv = (dstv0, dstv1)
    rows = (rows0, rows1)
    semg = (semg0, semg1)
    sems = (sems0, sems1)

    def load_group(g):
        base = t * CH + g * GRP
        pltpu.sync_copy(src_hbm.at[pl.ds(base, GRP)], srcv[g & 1])
        pltpu.sync_copy(dst_hbm.at[pl.ds(base, GRP)], dstv[g & 1])

    def sidx(j):
        g, r = divmod(j, GRP)
        return srcv[g & 1].at[r]

    def didx(j):
        g, r = divmod(j, GRP)
        return dstv[g & 1].at[r]

    def start_g(j):
        b = j & 1
        pltpu.async_copy(h_hbm.at[sidx(j)], rows[b], semg[b])

    def wait_g(j):
        b = j & 1
        pltpu.make_async_copy(h_hbm.at[sidx(j)], rows[b], semg[b]).wait()

    def start_s(j):
        b = j & 1
        pltpu.async_copy(rows[b], agg_sh.at[didx(j)], sems[b], add=True)

    def wait_s(j):
        b = j & 1
        pltpu.make_async_copy(rows[b], agg_sh.at[didx(j)], sems[b]).wait()

    load_group(0)
    start_g(0)
    start_g(1)
    wait_g(0)
    start_s(0)
    for p in range(1, CH // 2):
        j0, j1 = 2 * p, 2 * p + 1
        if j0 % GRP == 0:
            load_group(j0 // GRP)
        wait_s(j0 - 2)
        start_g(j0)
        wait_g(j0 - 1)
        start_s(j0 - 1)
        wait_s(j0 - 1)
        start_g(j1)
        wait_g(j0)
        start_s(j0)
    wait_g(CH - 1)
    start_s(CH - 1)
    wait_s(CH - 2)
    wait_s(CH - 1)
    plsc.subcore_barrier()

    def wout(j, carry):
        sl = pl.ds(r0 + j * 128, 128)
        pltpu.sync_copy(agg_sh.at[sl], rows0)
        pltpu.sync_copy(rows0, out_hbm.at[c, sl])
        return carry

    lax.fori_loop(0, RC, wout, 0)


def _pre1_body(cnt_ref, f_ref, o_ref):
    cout = cnt_ref[0][:, :1]
    dout = lax.rsqrt(jnp.maximum(cout, 1.0))
    o_ref[...] = f_ref[...] * dout


_CNT_SPEC = pl.BlockSpec((2, 128, 128), lambda i: (0, i, 0))
_ROW_SPEC = pl.BlockSpec((128, D), lambda i: (i, 0))
_AGG_SPEC = pl.BlockSpec((2, 128, D), lambda i: (0, i, 0))
_W_SPEC = pl.BlockSpec((D, D), lambda i: (0, 0))
_V_SPEC = pl.BlockSpec((1, D), lambda i: (0, 0))

_pre1 = pl.pallas_call(
    _pre1_body,
    grid=(NBLK,),
    in_specs=[_CNT_SPEC, _ROW_SPEC],
    out_specs=_ROW_SPEC,
    out_shape=jax.ShapeDtypeStruct((N_P, D), jnp.float32),
)


def _make_post(pool):
    def body(x_ref, aggp_ref, cnt_ref, W_ref, Wr_ref, b_ref, br_ref, g_ref, be_ref, *outs):
        i = pl.program_id(0)
        cnt = cnt_ref[...]
        cin = cnt[1][:, :1]
        din = lax.rsqrt(jnp.maximum(cin, 1.0))
        agg = (aggp_ref[0] + aggp_ref[1]) * din
        new = jnp.dot(agg, W_ref[...], preferred_element_type=jnp.float32) + b_ref[...]
        res = jnp.dot(x_ref[...], Wr_ref[...], preferred_element_type=jnp.float32) + br_ref[...]
        res = jnp.maximum(res, 0.0)
        h = (new + res) * BN_S * g_ref[...] + be_ref[...]
        if pool:
            (pool_ref,) = outs
            rows = i * 128 + lax.broadcasted_iota(jnp.int32, (128, 1), 0)
            ps = jnp.sum(jnp.where(rows < N, h, 0.0), axis=0, keepdims=True)

            @pl.when(i == 0)
            def _():
                pool_ref[...] = ps

            @pl.when(i > 0)
            def _():
                pool_ref[...] += ps
        else:
            h_ref, hs_ref = outs
            h_ref[...] = h
            cout = cnt[0][:, :1]
            dout = lax.rsqrt(jnp.maximum(cout, 1.0))
            hs_ref[...] = h * dout

    return body


_POST_IN_SPECS = [_ROW_SPEC, _AGG_SPEC, _CNT_SPEC,
                  _W_SPEC, _W_SPEC, _V_SPEC, _V_SPEC, _V_SPEC, _V_SPEC]

_post1 = pl.pallas_call(
    _make_post(False),
    grid=(NBLK,),
    in_specs=_POST_IN_SPECS,
    out_specs=[_ROW_SPEC, _ROW_SPEC],
    out_shape=[jax.ShapeDtypeStruct((N_P, D), jnp.float32),
               jax.ShapeDtypeStruct((N_P, D), jnp.float32)],
)

_post2 = pl.pallas_call(
    _make_post(True),
    grid=(NBLK,),
    in_specs=_POST_IN_SPECS,
    out_specs=pl.BlockSpec((1, D), lambda i: (0, 0)),
    out_shape=jax.ShapeDtypeStruct((1, D), jnp.float32),
)


def _head_body(pool_ref, s_ref, Wm1_ref, bm1_ref, gm1_ref, bem1_ref,
               Wm2_ref, bm2_ref, gm2_ref, bem2_ref, Wf_ref, bf_ref, o_ref):
    hg = jnp.maximum(pool_ref[...] * (1.0 / N), 0.0)
    x = jnp.dot(hg, Wm1_ref[0:D, :], preferred_element_type=jnp.float32)
    x = x + jnp.dot(s_ref[...], Wm1_ref[D:D + S_DIM, :], preferred_element_type=jnp.float32)
    x = jnp.maximum(x + bm1_ref[...], 0.0) * BN_S * gm1_ref[...] + bem1_ref[...]
    x = jnp.dot(x, Wm2_ref[...], preferred_element_type=jnp.float32) + bm2_ref[...]
    x = jnp.maximum(x, 0.0) * BN_S * gm2_ref[...] + bem2_ref[...]
    o = jnp.dot(x, Wf_ref[...], preferred_element_type=jnp.float32) + bf_ref[...]
    o_ref[...] = jnp.maximum(o, 0.0)


_head = pl.pallas_call(
    _head_body,
    out_shape=jax.ShapeDtypeStruct((1, NPRED), jnp.float32),
)


def kernel(edge_index, f, ef, s,
           W1, b1, Wr1, br1, g1, be1,
           W2, b2, Wr2, br2, g2, be2,
           Wm1, bm1, gm1, bem1,
           Wm2, bm2, gm2, bem2,
           Wf, bf):
    pad = (N + (jnp.arange(E_P - E, dtype=jnp.int32) % (N_P - N))).astype(jnp.int32)
    src_p = jnp.concatenate([edge_index[0], pad]).reshape(ROWS2D, 128)
    dst_p = jnp.concatenate([edge_index[1], pad]).reshape(ROWS2D, 128)
    f_p = jnp.pad(f, ((0, N_P - N), (0, 0)))

    cnt = _degrees(jnp.concatenate([src_p, dst_p], axis=0))
    h1s = _pre1(cnt, f_p)
    aggp1 = _msgpass(h1s, src_p, dst_p)
    h1, h2s = _post1(f_p, aggp1, cnt,
                     W1, Wr1, b1.reshape(1, D), br1.reshape(1, D),
                     g1.reshape(1, D), be1.reshape(1, D))
    aggp2 = _msgpass(h2s, src_p, dst_p)
    pool = _post2(h1, aggp2, cnt,
                  W2, Wr2, b2.reshape(1, D), br2.reshape(1, D),
                  g2.reshape(1, D), be2.reshape(1, D))
    out = _head(pool, s, Wm1, bm1.reshape(1, MLP_DIM), gm1.reshape(1, MLP_DIM),
                bem1.reshape(1, MLP_DIM), Wm2, bm2.reshape(1, MLP_DIM),
                gm2.reshape(1, MLP_DIM), bem2.reshape(1, MLP_DIM),
                Wf, bf.reshape(1, NPRED))
    return out


# speedup vs baseline: 8.6892x; 1.2190x over previous
import functools
import math

import jax
import jax.numpy as jnp
from jax import lax
from jax.experimental import pallas as pl
from jax.experimental.pallas import tpu as pltpu
from jax.experimental.pallas import tpu_sc as plsc

N = 10000
E = 320000
D = 128
S_DIM = 16
MLP_DIM = 128
NPRED = 1000
EPS = 1e-5
BN_S = 1.0 / math.sqrt(1.0 + EPS)

N_P = 10240
NBLK = N_P // 128
CH = 80
GRP = 16
EPT = CH * 128
E_P = 32 * EPT
ROWS2D = E_P // 128
RPT = N_P // 16
RC = RPT // 128

_MESH = plsc.VectorSubcoreMesh(core_axis_name="c", subcore_axis_name="s")


RPT_D = ROWS2D // 16
GRPD = 16
NGD = RPT_D // GRPD


@functools.partial(
    pl.kernel,
    out_type=jax.ShapeDtypeStruct((2, N_P, 128), jnp.float32),
    mesh=_MESH,
    scratch_types=[
        pltpu.VMEM((GRPD, 128), jnp.int32),
        pltpu.VMEM((128, 128), jnp.float32),
        pltpu.VMEM_SHARED((N_P, 128), jnp.float32),
    ],
)
def _degrees(idx_hbm, out_hbm, idxv, ones_v, acc_sh):
    c = lax.axis_index("c")
    s = lax.axis_index("s")
    r0 = s * RPT

    def fill0(i, carry):
        for k in range(8):
            ones_v[i, pl.ds(k * 16, 16)] = jnp.zeros((16,), jnp.float32)
        return carry

    lax.fori_loop(0, 128, fill0, 0)

    def zinit(j, carry):
        pltpu.sync_copy(ones_v, acc_sh.at[pl.ds(r0 + j * 128, 128)])
        return carry

    lax.fori_loop(0, RC, zinit, 0)

    def fill1(i, carry):
        for k in range(8):
            ones_v[i, pl.ds(k * 16, 16)] = jnp.float32(1.0) + jnp.zeros((16,), jnp.float32)
        return carry

    lax.fori_loop(0, 128, fill1, 0)
    plsc.subcore_barrier()

    def group(gi, carry):
        base = c * ROWS2D + s * RPT_D + gi * GRPD
        pltpu.sync_copy(idx_hbm.at[pl.ds(base, GRPD)], idxv)

        def step(j, carry2):
            pltpu.sync_copy(ones_v, acc_sh.at[idxv.at[j]], add=True)
            return carry2

        lax.fori_loop(0, GRPD, step, 0)
        return carry

    lax.fori_loop(0, NGD, group, 0)
    plsc.subcore_barrier()

    def wout(j, carry):
        sl = pl.ds(r0 + j * 128, 128)
        pltpu.sync_copy(acc_sh.at[sl], ones_v)
        pltpu.sync_copy(ones_v, out_hbm.at[c, sl])
        return carry

    lax.fori_loop(0, RC, wout, 0)


@functools.partial(
    pl.kernel,
    out_type=jax.ShapeDtypeStruct((2, N_P, D), jnp.float32),
    mesh=_MESH,
    scratch_types=[
        pltpu.VMEM((GRP, 128), jnp.int32),
        pltpu.VMEM((GRP, 128), jnp.int32),
        pltpu.VMEM((GRP, 128), jnp.int32),
        pltpu.VMEM((GRP, 128), jnp.int32),
        pltpu.VMEM((128, D), jnp.float32),
        pltpu.VMEM((128, D), jnp.float32),
        pltpu.VMEM_SHARED((N_P, D), jnp.float32),
        pltpu.SemaphoreType.DMA,
        pltpu.SemaphoreType.DMA,
        pltpu.SemaphoreType.DMA,
        pltpu.SemaphoreType.DMA,
    ],
)
def _msgpass(h_hbm, src_hbm, dst_hbm, out_hbm,
             srcv0, srcv1, dstv0, dstv1, rows0, rows1, agg_sh,
             semg0, semg1, sems0, sems1):
    c = lax.axis_index("c")
    s = lax.axis_index("s")
    t = c * 16 + s
    r0 = s * RPT

    def fill(i, carry):
        for k in range(8):
            rows0[i, pl.ds(k * 16, 16)] = jnp.zeros((16,), jnp.float32)
        return carry

    lax.fori_loop(0, 128, fill, 0)

    def zinit(j, carry):
        pltpu.sync_copy(rows0, agg_sh.at[pl.ds(r0 + j * 128, 128)])
        return carry

    lax.fori_loop(0, RC, zinit, 0)
    plsc.subcore_barrier()

    srcv = (srcv0, srcv1)
    dstv = (dstv0, dstv1)
    rows = (rows0, rows1)
    semg = (semg0, semg1)
    sems = (sems0, sems1)

    def load_group(g):
        base = t * CH + g * GRP
        pltpu.sync_copy(src_hbm.at[pl.ds(base, GRP)], srcv[g & 1])
        pltpu.sync_copy(dst_hbm.at[pl.ds(base, GRP)], dstv[g & 1])

    def sidx(j):
        g, r = divmod(j, GRP)
        return srcv[g & 1].at[r]

    def didx(j):
        g, r = divmod(j, GRP)
        return dstv[g & 1].at[r]

    def start_g(j):
        b = j & 1
        pltpu.async_copy(h_hbm.at[sidx(j)], rows[b], semg[b])

    def wait_g(j):
        b = j & 1
        pltpu.make_async_copy(h_hbm.at[sidx(j)], rows[b], semg[b]).wait()

    def start_s(j):
        b = j & 1
        pltpu.async_copy(rows[b], agg_sh.at[didx(j)], sems[b], add=True)

    def wait_s(j):
        b = j & 1
        pltpu.make_async_copy(rows[b], agg_sh.at[didx(j)], sems[b]).wait()

    load_group(0)
    start_g(0)
    start_g(1)
    wait_g(0)
    start_s(0)
    for p in range(1, CH // 2):
        j0, j1 = 2 * p, 2 * p + 1
        if j0 % GRP == 0:
            load_group(j0 // GRP)
        wait_s(j0 - 2)
        start_g(j0)
        wait_g(j0 - 1)
        start_s(j0 - 1)
        wait_s(j0 - 1)
        start_g(j1)
        wait_g(j0)
        start_s(j0)
    wait_g(CH - 1)
    start_s(CH - 1)
    wait_s(CH - 2)
    wait_s(CH - 1)
    plsc.subcore_barrier()

    def wout(j, carry):
        sl = pl.ds(r0 + j * 128, 128)
        pltpu.sync_copy(agg_sh.at[sl], rows0)
        pltpu.sync_copy(rows0, out_hbm.at[c, sl])
        return carry

    lax.fori_loop(0, RC, wout, 0)


RBLK = 512
NGRID = N_P // RBLK


def _pre1_body(cnt_ref, f_ref, o_ref):
    cout = cnt_ref[0][:, :1]
    dout = lax.rsqrt(jnp.maximum(cout, 1.0))
    o_ref[...] = f_ref[...] * dout


_CNT_SPEC = pl.BlockSpec((2, RBLK, 128), lambda i: (0, i, 0))
_CNT0_SPEC = pl.BlockSpec((1, RBLK, 128), lambda i: (0, i, 0))
_ROW_SPEC = pl.BlockSpec((RBLK, D), lambda i: (i, 0))
_AGG_SPEC = pl.BlockSpec((2, RBLK, D), lambda i: (0, i, 0))
_W_SPEC = pl.BlockSpec((D, D), lambda i: (0, 0))
_V_SPEC = pl.BlockSpec((1, D), lambda i: (0, 0))

_pre1 = pl.pallas_call(
    _pre1_body,
    grid=(NGRID,),
    in_specs=[_CNT0_SPEC, _ROW_SPEC],
    out_specs=_ROW_SPEC,
    out_shape=jax.ShapeDtypeStruct((N_P, D), jnp.float32),
)


def _make_post(pool):
    def body(x_ref, aggp_ref, cnt_ref, W_ref, Wr_ref, b_ref, br_ref, g_ref, be_ref, *outs):
        i = pl.program_id(0)
        cnt = cnt_ref[...]
        cin = cnt[1][:, :1]
        din = lax.rsqrt(jnp.maximum(cin, 1.0))
        agg = (aggp_ref[0] + aggp_ref[1]) * din
        new = jnp.dot(agg, W_ref[...], preferred_element_type=jnp.float32) + b_ref[...]
        res = jnp.dot(x_ref[...], Wr_ref[...], preferred_element_type=jnp.float32) + br_ref[...]
        res = jnp.maximum(res, 0.0)
        h = (new + res) * BN_S * g_ref[...] + be_ref[...]
        if pool:
            (pool_ref,) = outs
            rows = i * RBLK + lax.broadcasted_iota(jnp.int32, (RBLK, 1), 0)
            ps = jnp.sum(jnp.where(rows < N, h, 0.0), axis=0, keepdims=True)

            @pl.when(i == 0)
            def _():
                pool_ref[...] = ps

            @pl.when(i > 0)
            def _():
                pool_ref[...] += ps
        else:
            h_ref, hs_ref = outs
            h_ref[...] = h
            cout = cnt[0][:, :1]
            dout = lax.rsqrt(jnp.maximum(cout, 1.0))
            hs_ref[...] = h * dout

    return body


_POST_IN_SPECS = [_ROW_SPEC, _AGG_SPEC, _CNT_SPEC,
                  _W_SPEC, _W_SPEC, _V_SPEC, _V_SPEC, _V_SPEC, _V_SPEC]

_post1 = pl.pallas_call(
    _make_post(False),
    grid=(NGRID,),
    in_specs=_POST_IN_SPECS,
    out_specs=[_ROW_SPEC, _ROW_SPEC],
    out_shape=[jax.ShapeDtypeStruct((N_P, D), jnp.float32),
               jax.ShapeDtypeStruct((N_P, D), jnp.float32)],
)

_post2 = pl.pallas_call(
    _make_post(True),
    grid=(NGRID,),
    in_specs=_POST_IN_SPECS,
    out_specs=pl.BlockSpec((1, D), lambda i: (0, 0)),
    out_shape=jax.ShapeDtypeStruct((1, D), jnp.float32),
)


def _head_body(pool_ref, s_ref, Wm1_ref, bm1_ref, gm1_ref, bem1_ref,
               Wm2_ref, bm2_ref, gm2_ref, bem2_ref, Wf_ref, bf_ref, o_ref):
    hg = jnp.maximum(pool_ref[...] * (1.0 / N), 0.0)
    x = jnp.dot(hg, Wm1_ref[0:D, :], preferred_element_type=jnp.float32)
    x = x + jnp.dot(s_ref[...], Wm1_ref[D:D + S_DIM, :], preferred_element_type=jnp.float32)
    x = jnp.maximum(x + bm1_ref[...], 0.0) * BN_S * gm1_ref[...] + bem1_ref[...]
    x = jnp.dot(x, Wm2_ref[...], preferred_element_type=jnp.float32) + bm2_ref[...]
    x = jnp.maximum(x, 0.0) * BN_S * gm2_ref[...] + bem2_ref[...]
    o = jnp.dot(x, Wf_ref[...], preferred_element_type=jnp.float32) + bf_ref[...]
    o_ref[...] = jnp.maximum(o, 0.0)


_head = pl.pallas_call(
    _head_body,
    out_shape=jax.ShapeDtypeStruct((1, NPRED), jnp.float32),
)


def kernel(edge_index, f, ef, s,
           W1, b1, Wr1, br1, g1, be1,
           W2, b2, Wr2, br2, g2, be2,
           Wm1, bm1, gm1, bem1,
           Wm2, bm2, gm2, bem2,
           Wf, bf):
    pad = (N + (jnp.arange(E_P - E, dtype=jnp.int32) % (N_P - N))).astype(jnp.int32)
    src_p = jnp.concatenate([edge_index[0], pad]).reshape(ROWS2D, 128)
    dst_p = jnp.concatenate([edge_index[1], pad]).reshape(ROWS2D, 128)
    f_p = jnp.pad(f, ((0, N_P - N), (0, 0)))

    cnt = _degrees(jnp.concatenate([src_p, dst_p], axis=0))
    h1s = _pre1(cnt, f_p)
    aggp1 = _msgpass(h1s, src_p, dst_p)
    h1, h2s = _post1(f_p, aggp1, cnt,
                     W1, Wr1, b1.reshape(1, D), br1.reshape(1, D),
                     g1.reshape(1, D), be1.reshape(1, D))
    aggp2 = _msgpass(h2s, src_p, dst_p)
    pool = _post2(h1, aggp2, cnt,
                  W2, Wr2, b2.reshape(1, D), br2.reshape(1, D),
                  g2.reshape(1, D), be2.reshape(1, D))
    out = _head(pool, s, Wm1, bm1.reshape(1, MLP_DIM), gm1.reshape(1, MLP_DIM),
                bem1.reshape(1, MLP_DIM), Wm2, bm2.reshape(1, MLP_DIM),
                gm2.reshape(1, MLP_DIM), bem2.reshape(1, MLP_DIM),
                Wf, bf.reshape(1, NPRED))
    return out


# speedup vs baseline: 9.0331x; 1.0396x over previous
import functools
import math

import jax
import jax.numpy as jnp
from jax import lax
from jax.experimental import pallas as pl
from jax.experimental.pallas import tpu as pltpu
from jax.experimental.pallas import tpu_sc as plsc

N = 10000
E = 320000
D = 128
S_DIM = 16
MLP_DIM = 128
NPRED = 1000
EPS = 1e-5
BN_S = 1.0 / math.sqrt(1.0 + EPS)

N_P = 10240
NBLK = N_P // 128
CH = 80
GRP = 16
EPT = CH * 128
E_P = 32 * EPT
ROWS2D = E_P // 128
RPT = N_P // 16
RC = RPT // 128

_MESH = plsc.VectorSubcoreMesh(core_axis_name="c", subcore_axis_name="s")


RPT_D = ROWS2D // 16
GRPD = 16
NGD = RPT_D // GRPD


@functools.partial(
    pl.kernel,
    out_type=jax.ShapeDtypeStruct((2, N_P, 128), jnp.float32),
    mesh=_MESH,
    scratch_types=[
        pltpu.VMEM((GRPD, 128), jnp.int32),
        pltpu.VMEM((128, 128), jnp.float32),
        pltpu.VMEM_SHARED((N_P, 128), jnp.float32),
    ],
)
def _degrees(idx_hbm, out_hbm, idxv, ones_v, acc_sh):
    c = lax.axis_index("c")
    s = lax.axis_index("s")
    r0 = s * RPT

    def fill0(i, carry):
        for k in range(8):
            ones_v[i, pl.ds(k * 16, 16)] = jnp.zeros((16,), jnp.float32)
        return carry

    lax.fori_loop(0, 128, fill0, 0)

    def zinit(j, carry):
        pltpu.sync_copy(ones_v, acc_sh.at[pl.ds(r0 + j * 128, 128)])
        return carry

    lax.fori_loop(0, RC, zinit, 0)

    def fill1(i, carry):
        for k in range(8):
            ones_v[i, pl.ds(k * 16, 16)] = jnp.float32(1.0) + jnp.zeros((16,), jnp.float32)
        return carry

    lax.fori_loop(0, 128, fill1, 0)
    plsc.subcore_barrier()

    def group(gi, carry):
        base = c * ROWS2D + s * RPT_D + gi * GRPD
        pltpu.sync_copy(idx_hbm.at[pl.ds(base, GRPD)], idxv)

        def step(j, carry2):
            pltpu.sync_copy(ones_v, acc_sh.at[idxv.at[j]], add=True)
            return carry2

        lax.fori_loop(0, GRPD, step, 0)
        return carry

    lax.fori_loop(0, NGD, group, 0)
    plsc.subcore_barrier()

    def wout(j, carry):
        sl = pl.ds(r0 + j * 128, 128)
        pltpu.sync_copy(acc_sh.at[sl], ones_v)
        pltpu.sync_copy(ones_v, out_hbm.at[c, sl])
        return carry

    lax.fori_loop(0, RC, wout, 0)


@functools.partial(
    pl.kernel,
    out_type=jax.ShapeDtypeStruct((2, N_P, D), jnp.float32),
    mesh=_MESH,
    scratch_types=[
        pltpu.VMEM((GRP, 128), jnp.int32),
        pltpu.VMEM((GRP, 128), jnp.int32),
        pltpu.VMEM((GRP, 128), jnp.int32),
        pltpu.VMEM((GRP, 128), jnp.int32),
        pltpu.VMEM((128, D), jnp.float32),
        pltpu.VMEM((128, D), jnp.float32),
        pltpu.VMEM_SHARED((N_P, D), jnp.float32),
        pltpu.SemaphoreType.DMA,
        pltpu.SemaphoreType.DMA,
        pltpu.SemaphoreType.DMA,
        pltpu.SemaphoreType.DMA,
    ],
)
def _msgpass(h_hbm, src_hbm, dst_hbm, out_hbm,
             srcv0, srcv1, dstv0, dstv1, rows0, rows1, agg_sh,
             semg0, semg1, sems0, sems1):
    c = lax.axis_index("c")
    s = lax.axis_index("s")
    t = c * 16 + s
    r0 = s * RPT

    def fill(i, carry):
        for k in range(8):
            rows0[i, pl.ds(k * 16, 16)] = jnp.zeros((16,), jnp.float32)
        return carry

    lax.fori_loop(0, 128, fill, 0)

    def zinit(j, carry):
        pltpu.sync_copy(rows0, agg_sh.at[pl.ds(r0 + j * 128, 128)])
        return carry

    lax.fori_loop(0, RC, zinit, 0)
    plsc.subcore_barrier()

    srcv = (srcv0, srcv1)
    dstv = (dstv0, dstv1)
    rows = (rows0, rows1)
    semg = (semg0, semg1)
    sems = (sems0, sems1)

    def load_group(g):
        base = t * CH + g * GRP
        pltpu.sync_copy(src_hbm.at[pl.ds(base, GRP)], srcv[g & 1])
        pltpu.sync_copy(dst_hbm.at[pl.ds(base, GRP)], dstv[g & 1])

    def sidx(j):
        g, r = divmod(j, GRP)
        return srcv[g & 1].at[r]

    def didx(j):
        g, r = divmod(j, GRP)
        return dstv[g & 1].at[r]

    def start_g(j):
        b = j & 1
        pltpu.async_copy(h_hbm.at[sidx(j)], rows[b], semg[b])

    def wait_g(j):
        b = j & 1
        pltpu.make_async_copy(h_hbm.at[sidx(j)], rows[b], semg[b]).wait()

    def start_s(j):
        b = j & 1
        pltpu.async_copy(rows[b], agg_sh.at[didx(j)], sems[b], add=True)

    def wait_s(j):
        b = j & 1
        pltpu.make_async_copy(rows[b], agg_sh.at[didx(j)], sems[b]).wait()

    load_group(0)
    start_g(0)
    start_g(1)
    wait_g(0)
    start_s(0)
    for p in range(1, CH // 2):
        j0, j1 = 2 * p, 2 * p + 1
        if j0 % GRP == 0:
            load_group(j0 // GRP)
        wait_s(j0 - 2)
        start_g(j0)
        wait_g(j0 - 1)
        start_s(j0 - 1)
        wait_s(j0 - 1)
        start_g(j1)
        wait_g(j0)
        start_s(j0)
    wait_g(CH - 1)
    start_s(CH - 1)
    wait_s(CH - 2)
    wait_s(CH - 1)
    plsc.subcore_barrier()

    def wout(j, carry):
        sl = pl.ds(r0 + j * 128, 128)
        pltpu.sync_copy(agg_sh.at[sl], rows0)
        pltpu.sync_copy(rows0, out_hbm.at[c, sl])
        return carry

    lax.fori_loop(0, RC, wout, 0)


RBLK = 1024
NGRID = N_P // RBLK


def _pre1_body(cnt_ref, f_ref, o_ref):
    cout = cnt_ref[0][:, :1]
    dout = lax.rsqrt(jnp.maximum(cout, 1.0))
    o_ref[...] = f_ref[...] * dout


_CNT_SPEC = pl.BlockSpec((2, RBLK, 128), lambda i: (0, i, 0))
_CNT0_SPEC = pl.BlockSpec((1, RBLK, 128), lambda i: (0, i, 0))
_ROW_SPEC = pl.BlockSpec((RBLK, D), lambda i: (i, 0))
_AGG_SPEC = pl.BlockSpec((2, RBLK, D), lambda i: (0, i, 0))
_W_SPEC = pl.BlockSpec((D, D), lambda i: (0, 0))
_V_SPEC = pl.BlockSpec((1, D), lambda i: (0, 0))

_pre1 = pl.pallas_call(
    _pre1_body,
    grid=(NGRID,),
    in_specs=[_CNT0_SPEC, _ROW_SPEC],
    out_specs=_ROW_SPEC,
    out_shape=jax.ShapeDtypeStruct((N_P, D), jnp.float32),
)


def _make_post(pool):
    def body(x_ref, aggp_ref, cnt_ref, W_ref, Wr_ref, b_ref, br_ref, g_ref, be_ref, *outs):
        i = pl.program_id(0)
        cnt = cnt_ref[...]
        cin = cnt[1][:, :1]
        din = lax.rsqrt(jnp.maximum(cin, 1.0))
        agg = (aggp_ref[0] + aggp_ref[1]) * din
        new = jnp.dot(agg, W_ref[...], preferred_element_type=jnp.float32) + b_ref[...]
        res = jnp.dot(x_ref[...], Wr_ref[...], preferred_element_type=jnp.float32) + br_ref[...]
        res = jnp.maximum(res, 0.0)
        h = (new + res) * BN_S * g_ref[...] + be_ref[...]
        if pool:
            (pool_ref,) = outs
            rows = i * RBLK + lax.broadcasted_iota(jnp.int32, (RBLK, 1), 0)
            ps = jnp.sum(jnp.where(rows < N, h, 0.0), axis=0, keepdims=True)

            @pl.when(i == 0)
            def _():
                pool_ref[...] = ps

            @pl.when(i > 0)
            def _():
                pool_ref[...] += ps
        else:
            h_ref, hs_ref = outs
            h_ref[...] = h
            cout = cnt[0][:, :1]
            dout = lax.rsqrt(jnp.maximum(cout, 1.0))
            hs_ref[...] = h * dout

    return body


_POST_IN_SPECS = [_ROW_SPEC, _AGG_SPEC, _CNT_SPEC,
                  _W_SPEC, _W_SPEC, _V_SPEC, _V_SPEC, _V_SPEC, _V_SPEC]

_post1 = pl.pallas_call(
    _make_post(False),
    grid=(NGRID,),
    in_specs=_POST_IN_SPECS,
    out_specs=[_ROW_SPEC, _ROW_SPEC],
    out_shape=[jax.ShapeDtypeStruct((N_P, D), jnp.float32),
               jax.ShapeDtypeStruct((N_P, D), jnp.float32)],
)

_post2 = pl.pallas_call(
    _make_post(True),
    grid=(NGRID,),
    in_specs=_POST_IN_SPECS,
    out_specs=pl.BlockSpec((1, D), lambda i: (0, 0)),
    out_shape=jax.ShapeDtypeStruct((1, D), jnp.float32),
)


def _head_body(pool_ref, s_ref, Wm1_ref, bm1_ref, gm1_ref, bem1_ref,
               Wm2_ref, bm2_ref, gm2_ref, bem2_ref, Wf_ref, bf_ref, o_ref):
    hg = jnp.maximum(pool_ref[...] * (1.0 / N), 0.0)
    x = jnp.dot(hg, Wm1_ref[0:D, :], preferred_element_type=jnp.float32)
    x = x + jnp.dot(s_ref[...], Wm1_ref[D:D + S_DIM, :], preferred_element_type=jnp.float32)
    x = jnp.maximum(x + bm1_ref[...], 0.0) * BN_S * gm1_ref[...] + bem1_ref[...]
    x = jnp.dot(x, Wm2_ref[...], preferred_element_type=jnp.float32) + bm2_ref[...]
    x = jnp.maximum(x, 0.0) * BN_S * gm2_ref[...] + bem2_ref[...]
    o = jnp.dot(x, Wf_ref[...], preferred_element_type=jnp.float32) + bf_ref[...]
    o_ref[...] = jnp.maximum(o, 0.0)


_head = pl.pallas_call(
    _head_body,
    out_shape=jax.ShapeDtypeStruct((1, NPRED), jnp.float32),
)


def kernel(edge_index, f, ef, s,
           W1, b1, Wr1, br1, g1, be1,
           W2, b2, Wr2, br2, g2, be2,
           Wm1, bm1, gm1, bem1,
           Wm2, bm2, gm2, bem2,
           Wf, bf):
    pad = (N + (jnp.arange(E_P - E, dtype=jnp.int32) % (N_P - N))).astype(jnp.int32)
    src_p = jnp.concatenate([edge_index[0], pad]).reshape(ROWS2D, 128)
    dst_p = jnp.concatenate([edge_index[1], pad]).reshape(ROWS2D, 128)
    f_p = jnp.pad(f, ((0, N_P - N), (0, 0)))

    cnt = _degrees(jnp.concatenate([src_p, dst_p], axis=0))
    h1s = _pre1(cnt, f_p)
    aggp1 = _msgpass(h1s, src_p, dst_p)
    h1, h2s = _post1(f_p, aggp1, cnt,
                     W1, Wr1, b1.reshape(1, D), br1.reshape(1, D),
                     g1.reshape(1, D), be1.reshape(1, D))
    aggp2 = _msgpass(h2s, src_p, dst_p)
    pool = _post2(h1, aggp2, cnt,
                  W2, Wr2, b2.reshape(1, D), br2.reshape(1, D),
                  g2.reshape(1, D), be2.reshape(1, D))
    out = _head(pool, s, Wm1, bm1.reshape(1, MLP_DIM), gm1.reshape(1, MLP_DIM),
                bem1.reshape(1, MLP_DIM), Wm2, bm2.reshape(1, MLP_DIM),
                gm2.reshape(1, MLP_DIM), bem2.reshape(1, MLP_DIM),
                Wf, bf.reshape(1, NPRED))
    return out


# speedup vs baseline: 9.2148x; 1.0201x over previous
import functools
import math

import jax
import jax.numpy as jnp
from jax import lax
from jax.experimental import pallas as pl
from jax.experimental.pallas import tpu as pltpu
from jax.experimental.pallas import tpu_sc as plsc

N = 10000
E = 320000
D = 128
S_DIM = 16
MLP_DIM = 128
NPRED = 1000
EPS = 1e-5
BN_S = 1.0 / math.sqrt(1.0 + EPS)

N_P = 10240
NBLK = N_P // 128
CH = 80
GRP = 16
EPT = CH * 128
E_P = 32 * EPT
ROWS2D = E_P // 128
RPT = N_P // 16
RC = RPT // 128

_MESH = plsc.VectorSubcoreMesh(core_axis_name="c", subcore_axis_name="s")


RPT_D = ROWS2D // 16
GRPD = 16
NGD = RPT_D // GRPD


@functools.partial(
    pl.kernel,
    out_type=jax.ShapeDtypeStruct((2, N_P, 128), jnp.float32),
    mesh=_MESH,
    scratch_types=[
        pltpu.VMEM((GRPD, 128), jnp.int32),
        pltpu.VMEM((GRPD, 128), jnp.int32),
        pltpu.VMEM((128, 128), jnp.float32),
        pltpu.VMEM_SHARED((N_P, 128), jnp.float32),
        pltpu.SemaphoreType.DMA,
        pltpu.SemaphoreType.DMA,
    ],
)
def _degrees(idx_hbm, out_hbm, idxv0, idxv1, ones_v, acc_sh, semd0, semd1):
    c = lax.axis_index("c")
    s = lax.axis_index("s")
    r0 = s * RPT

    def fill0(i, carry):
        for k in range(8):
            ones_v[i, pl.ds(k * 16, 16)] = jnp.zeros((16,), jnp.float32)
        return carry

    lax.fori_loop(0, 128, fill0, 0)

    def zinit(j, carry):
        pltpu.sync_copy(ones_v, acc_sh.at[pl.ds(r0 + j * 128, 128)])
        return carry

    lax.fori_loop(0, RC, zinit, 0)

    def fill1(i, carry):
        for k in range(8):
            ones_v[i, pl.ds(k * 16, 16)] = jnp.float32(1.0) + jnp.zeros((16,), jnp.float32)
        return carry

    lax.fori_loop(0, 128, fill1, 0)
    plsc.subcore_barrier()

    idxv = (idxv0, idxv1)
    semd = (semd0, semd1)

    def load_group(g):
        base = c * ROWS2D + s * RPT_D + g * GRPD
        pltpu.sync_copy(idx_hbm.at[pl.ds(base, GRPD)], idxv[g & 1])

    def fire_group(g):
        for j in range(GRPD):
            pltpu.async_copy(ones_v, acc_sh.at[idxv[g & 1].at[j]], semd[g & 1], add=True)

    def drain_group(g):
        for j in range(GRPD):
            pltpu.make_async_copy(ones_v, acc_sh.at[idxv[g & 1].at[j]], semd[g & 1]).wait()

    load_group(0)
    fire_group(0)
    load_group(1)
    fire_group(1)
    for g in range(2, NGD):
        drain_group(g - 2)
        load_group(g)
        fire_group(g)
    drain_group(NGD - 2)
    drain_group(NGD - 1)
    plsc.subcore_barrier()

    def wout(j, carry):
        sl = pl.ds(r0 + j * 128, 128)
        pltpu.sync_copy(acc_sh.at[sl], ones_v)
        pltpu.sync_copy(ones_v, out_hbm.at[c, sl])
        return carry

    lax.fori_loop(0, RC, wout, 0)


@functools.partial(
    pl.kernel,
    out_type=jax.ShapeDtypeStruct((2, N_P, D), jnp.float32),
    mesh=_MESH,
    scratch_types=[
        pltpu.VMEM((GRP, 128), jnp.int32),
        pltpu.VMEM((GRP, 128), jnp.int32),
        pltpu.VMEM((GRP, 128), jnp.int32),
        pltpu.VMEM((GRP, 128), jnp.int32),
        pltpu.VMEM((128, D), jnp.float32),
        pltpu.VMEM((128, D), jnp.float32),
        pltpu.VMEM_SHARED((N_P, D), jnp.float32),
        pltpu.SemaphoreType.DMA,
        pltpu.SemaphoreType.DMA,
        pltpu.SemaphoreType.DMA,
        pltpu.SemaphoreType.DMA,
    ],
)
def _msgpass(h_hbm, src_hbm, dst_hbm, out_hbm,
             srcv0, srcv1, dstv0, dstv1, rows0, rows1, agg_sh,
             semg0, semg1, sems0, sems1):
    c = lax.axis_index("c")
    s = lax.axis_index("s")
    t = c * 16 + s
    r0 = s * RPT

    def fill(i, carry):
        for k in range(8):
            rows0[i, pl.ds(k * 16, 16)] = jnp.zeros((16,), jnp.float32)
        return carry

    lax.fori_loop(0, 128, fill, 0)

    def zinit(j, carry):
        pltpu.sync_copy(rows0, agg_sh.at[pl.ds(r0 + j * 128, 128)])
        return carry

    lax.fori_loop(0, RC, zinit, 0)
    plsc.subcore_barrier()

    srcv = (srcv0, srcv1)
    dstv = (dstv0, dstv1)
    rows = (rows0, rows1)
    semg = (semg0, semg1)
    sems = (sems0, sems1)

    def load_group(g):
        base = t * CH + g * GRP
        pltpu.sync_copy(src_hbm.at[pl.ds(base, GRP)], srcv[g & 1])
        pltpu.sync_copy(dst_hbm.at[pl.ds(base, GRP)], dstv[g & 1])

    def sidx(j):
        g, r = divmod(j, GRP)
        return srcv[g & 1].at[r]

    def didx(j):
        g, r = divmod(j, GRP)
        return dstv[g & 1].at[r]

    def start_g(j):
        b = j & 1
        pltpu.async_copy(h_hbm.at[sidx(j)], rows[b], semg[b])

    def wait_g(j):
        b = j & 1
        pltpu.make_async_copy(h_hbm.at[sidx(j)], rows[b], semg[b]).wait()

    def start_s(j):
        b = j & 1
        pltpu.async_copy(rows[b], agg_sh.at[didx(j)], sems[b], add=True)

    def wait_s(j):
        b = j & 1
        pltpu.make_async_copy(rows[b], agg_sh.at[didx(j)], sems[b]).wait()

    load_group(0)
    start_g(0)
    start_g(1)
    wait_g(0)
    start_s(0)
    for p in range(1, CH // 2):
        j0, j1 = 2 * p, 2 * p + 1
        if j0 % GRP == 0:
            load_group(j0 // GRP)
        wait_s(j0 - 2)
        start_g(j0)
        wait_g(j0 - 1)
        start_s(j0 - 1)
        wait_s(j0 - 1)
        start_g(j1)
        wait_g(j0)
        start_s(j0)
    wait_g(CH - 1)
    start_s(CH - 1)
    wait_s(CH - 2)
    wait_s(CH - 1)
    plsc.subcore_barrier()

    def wout(j, carry):
        sl = pl.ds(r0 + j * 128, 128)
        pltpu.sync_copy(agg_sh.at[sl], rows0)
        pltpu.sync_copy(rows0, out_hbm.at[c, sl])
        return carry

    lax.fori_loop(0, RC, wout, 0)


RBLK = 1024
NGRID = N_P // RBLK


def _pre1_body(cnt_ref, f_ref, o_ref):
    cout = cnt_ref[0][:, :1]
    dout = lax.rsqrt(jnp.maximum(cout, 1.0))
    o_ref[...] = f_ref[...] * dout


_CNT_SPEC = pl.BlockSpec((2, RBLK, 128), lambda i: (0, i, 0))
_CNT0_SPEC = pl.BlockSpec((1, RBLK, 128), lambda i: (0, i, 0))
_ROW_SPEC = pl.BlockSpec((RBLK, D), lambda i: (i, 0))
_AGG_SPEC = pl.BlockSpec((2, RBLK, D), lambda i: (0, i, 0))
_W_SPEC = pl.BlockSpec((D, D), lambda i: (0, 0))
_V_SPEC = pl.BlockSpec((1, D), lambda i: (0, 0))

_pre1 = pl.pallas_call(
    _pre1_body,
    grid=(NGRID,),
    in_specs=[_CNT0_SPEC, _ROW_SPEC],
    out_specs=_ROW_SPEC,
    out_shape=jax.ShapeDtypeStruct((N_P, D), jnp.float32),
)


def _make_post(pool):
    def body(x_ref, aggp_ref, cnt_ref, W_ref, Wr_ref, b_ref, br_ref, g_ref, be_ref, *outs):
        i = pl.program_id(0)
        cnt = cnt_ref[...]
        cin = cnt[1][:, :1]
        din = lax.rsqrt(jnp.maximum(cin, 1.0))
        agg = (aggp_ref[0] + aggp_ref[1]) * din
        new = jnp.dot(agg, W_ref[...], preferred_element_type=jnp.float32) + b_ref[...]
        res = jnp.dot(x_ref[...], Wr_ref[...], preferred_element_type=jnp.float32) + br_ref[...]
        res = jnp.maximum(res, 0.0)
        h = (new + res) * BN_S * g_ref[...] + be_ref[...]
        if pool:
            (pool_ref,) = outs
            rows = i * RBLK + lax.broadcasted_iota(jnp.int32, (RBLK, 1), 0)
            ps = jnp.sum(jnp.where(rows < N, h, 0.0), axis=0, keepdims=True)

            @pl.when(i == 0)
            def _():
                pool_ref[...] = ps

            @pl.when(i > 0)
            def _():
                pool_ref[...] += ps
        else:
            h_ref, hs_ref = outs
            h_ref[...] = h
            cout = cnt[0][:, :1]
            dout = lax.rsqrt(jnp.maximum(cout, 1.0))
            hs_ref[...] = h * dout

    return body


_POST_IN_SPECS = [_ROW_SPEC, _AGG_SPEC, _CNT_SPEC,
                  _W_SPEC, _W_SPEC, _V_SPEC, _V_SPEC, _V_SPEC, _V_SPEC]

_post1 = pl.pallas_call(
    _make_post(False),
    grid=(NGRID,),
    in_specs=_POST_IN_SPECS,
    out_specs=[_ROW_SPEC, _ROW_SPEC],
    out_shape=[jax.ShapeDtypeStruct((N_P, D), jnp.float32),
               jax.ShapeDtypeStruct((N_P, D), jnp.float32)],
)

_post2 = pl.pallas_call(
    _make_post(True),
    grid=(NGRID,),
    in_specs=_POST_IN_SPECS,
    out_specs=pl.BlockSpec((1, D), lambda i: (0, 0)),
    out_shape=jax.ShapeDtypeStruct((1, D), jnp.float32),
)


def _head_body(pool_ref, s_ref, Wm1_ref, bm1_ref, gm1_ref, bem1_ref,
               Wm2_ref, bm2_ref, gm2_ref, bem2_ref, Wf_ref, bf_ref, o_ref):
    hg = jnp.maximum(pool_ref[...] * (1.0 / N), 0.0)
    x = jnp.dot(hg, Wm1_ref[0:D, :], preferred_element_type=jnp.float32)
    x = x + jnp.dot(s_ref[...], Wm1_ref[D:D + S_DIM, :], preferred_element_type=jnp.float32)
    x = jnp.maximum(x + bm1_ref[...], 0.0) * BN_S * gm1_ref[...] + bem1_ref[...]
    x = jnp.dot(x, Wm2_ref[...], preferred_element_type=jnp.float32) + bm2_ref[...]
    x = jnp.maximum(x, 0.0) * BN_S * gm2_ref[...] + bem2_ref[...]
    o = jnp.dot(x, Wf_ref[...], preferred_element_type=jnp.float32) + bf_ref[...]
    o_ref[...] = jnp.maximum(o, 0.0)


_head = pl.pallas_call(
    _head_body,
    out_shape=jax.ShapeDtypeStruct((1, NPRED), jnp.float32),
)


def kernel(edge_index, f, ef, s,
           W1, b1, Wr1, br1, g1, be1,
           W2, b2, Wr2, br2, g2, be2,
           Wm1, bm1, gm1, bem1,
           Wm2, bm2, gm2, bem2,
           Wf, bf):
    pad = (N + (jnp.arange(E_P - E, dtype=jnp.int32) % (N_P - N))).astype(jnp.int32)
    src_p = jnp.concatenate([edge_index[0], pad]).reshape(ROWS2D, 128)
    dst_p = jnp.concatenate([edge_index[1], pad]).reshape(ROWS2D, 128)
    f_p = jnp.pad(f, ((0, N_P - N), (0, 0)))

    cnt = _degrees(jnp.concatenate([src_p, dst_p], axis=0))
    h1s = _pre1(cnt, f_p)
    aggp1 = _msgpass(h1s, src_p, dst_p)
    h1, h2s = _post1(f_p, aggp1, cnt,
                     W1, Wr1, b1.reshape(1, D), br1.reshape(1, D),
                     g1.reshape(1, D), be1.reshape(1, D))
    aggp2 = _msgpass(h2s, src_p, dst_p)
    pool = _post2(h1, aggp2, cnt,
                  W2, Wr2, b2.reshape(1, D), br2.reshape(1, D),
                  g2.reshape(1, D), be2.reshape(1, D))
    out = _head(pool, s, Wm1, bm1.reshape(1, MLP_DIM), gm1.reshape(1, MLP_DIM),
                bem1.reshape(1, MLP_DIM), Wm2, bm2.reshape(1, MLP_DIM),
                gm2.reshape(1, MLP_DIM), bem2.reshape(1, MLP_DIM),
                Wf, bf.reshape(1, NPRED))
    return out


# speedup vs baseline: 9.3161x; 1.0110x over previous
import functools
import math

import jax
import jax.numpy as jnp
from jax import lax
from jax.experimental import pallas as pl
from jax.experimental.pallas import tpu as pltpu
from jax.experimental.pallas import tpu_sc as plsc

N = 10000
E = 320000
D = 128
S_DIM = 16
MLP_DIM = 128
NPRED = 1000
EPS = 1e-5
BN_S = 1.0 / math.sqrt(1.0 + EPS)

N_P = 10240
NBLK = N_P // 128
CH = 80
GRP = 16
EPT = CH * 128
E_P = 32 * EPT
ROWS2D = E_P // 128
RPT = N_P // 16
RC = RPT // 128

_MESH = plsc.VectorSubcoreMesh(core_axis_name="c", subcore_axis_name="s")


RPT_D = ROWS2D // 16
GRPD = 16
NGD = RPT_D // GRPD


@functools.partial(
    pl.kernel,
    out_type=jax.ShapeDtypeStruct((2, N_P, 128), jnp.float32),
    mesh=_MESH,
    scratch_types=[
        pltpu.VMEM((GRPD, 128), jnp.int32),
        pltpu.VMEM((GRPD, 128), jnp.int32),
        pltpu.VMEM((128, 128), jnp.float32),
        pltpu.VMEM_SHARED((N_P, 128), jnp.float32),
        pltpu.SemaphoreType.DMA,
        pltpu.SemaphoreType.DMA,
    ],
)
def _degrees(idx_hbm, out_hbm, idxv0, idxv1, ones_v, acc_sh, semd0, semd1):
    c = lax.axis_index("c")
    s = lax.axis_index("s")
    r0 = s * RPT

    def fill0(i, carry):
        for k in range(8):
            ones_v[i, pl.ds(k * 16, 16)] = jnp.zeros((16,), jnp.float32)
        return carry

    lax.fori_loop(0, 128, fill0, 0)

    def zinit(j, carry):
        pltpu.sync_copy(ones_v, acc_sh.at[pl.ds(r0 + j * 128, 128)])
        return carry

    lax.fori_loop(0, RC, zinit, 0)

    def fill1(i, carry):
        for k in range(8):
            ones_v[i, pl.ds(k * 16, 16)] = jnp.float32(1.0) + jnp.zeros((16,), jnp.float32)
        return carry

    lax.fori_loop(0, 128, fill1, 0)
    plsc.subcore_barrier()

    idxv = (idxv0, idxv1)
    semd = (semd0, semd1)

    def load_group(g):
        base = c * ROWS2D + s * RPT_D + g * GRPD
        pltpu.sync_copy(idx_hbm.at[pl.ds(base, GRPD)], idxv[g & 1])

    def fire_group(g):
        for j in range(GRPD):
            pltpu.async_copy(ones_v, acc_sh.at[idxv[g & 1].at[j]], semd[g & 1], add=True)

    def drain_group(g):
        for j in range(GRPD):
            pltpu.make_async_copy(ones_v, acc_sh.at[idxv[g & 1].at[j]], semd[g & 1]).wait()

    load_group(0)
    fire_group(0)
    load_group(1)
    fire_group(1)
    for g in range(2, NGD):
        drain_group(g - 2)
        load_group(g)
        fire_group(g)
    drain_group(NGD - 2)
    drain_group(NGD - 1)
    plsc.subcore_barrier()

    def wout(j, carry):
        sl = pl.ds(r0 + j * 128, 128)
        pltpu.sync_copy(acc_sh.at[sl], ones_v)
        pltpu.sync_copy(ones_v, out_hbm.at[c, sl])
        return carry

    lax.fori_loop(0, RC, wout, 0)


@functools.partial(
    pl.kernel,
    out_type=jax.ShapeDtypeStruct((2, N_P, D), jnp.float32),
    mesh=_MESH,
    scratch_types=[
        pltpu.VMEM((GRP, 128), jnp.int32),
        pltpu.VMEM((GRP, 128), jnp.int32),
        pltpu.VMEM((GRP, 128), jnp.int32),
        pltpu.VMEM((GRP, 128), jnp.int32),
        pltpu.VMEM((128, D), jnp.float32),
        pltpu.VMEM((128, D), jnp.float32),
        pltpu.VMEM_SHARED((N_P, D), jnp.float32),
        pltpu.SemaphoreType.DMA,
        pltpu.SemaphoreType.DMA,
        pltpu.SemaphoreType.DMA,
        pltpu.SemaphoreType.DMA,
    ],
)
def _msgpass(h_hbm, src_hbm, dst_hbm, out_hbm,
             srcv0, srcv1, dstv0, dstv1, rows0, rows1, agg_sh,
             semg0, semg1, sems0, sems1):
    c = lax.axis_index("c")
    s = lax.axis_index("s")
    t = c * 16 + s
    r0 = s * RPT

    def fill(i, carry):
        for k in range(8):
            rows0[i, pl.ds(k * 16, 16)] = jnp.zeros((16,), jnp.float32)
        return carry

    lax.fori_loop(0, 128, fill, 0)

    def zinit(j, carry):
        pltpu.sync_copy(rows0, agg_sh.at[pl.ds(r0 + j * 128, 128)])
        return carry

    lax.fori_loop(0, RC, zinit, 0)
    plsc.subcore_barrier()

    srcv = (srcv0, srcv1)
    dstv = (dstv0, dstv1)
    rows = (rows0, rows1)
    semg = (semg0, semg1)
    sems = (sems0, sems1)

    def load_group(g):
        base = t * CH + g * GRP
        pltpu.sync_copy(src_hbm.at[pl.ds(base, GRP)], srcv[g & 1])
        pltpu.sync_copy(dst_hbm.at[pl.ds(base, GRP)], dstv[g & 1])

    def sidx(j):
        g, r = divmod(j, GRP)
        return srcv[g & 1].at[r]

    def didx(j):
        g, r = divmod(j, GRP)
        return dstv[g & 1].at[r]

    def start_g(j):
        b = j & 1
        pltpu.async_copy(h_hbm.at[sidx(j)], rows[b], semg[b])

    def wait_g(j):
        b = j & 1
        pltpu.make_async_copy(h_hbm.at[sidx(j)], rows[b], semg[b]).wait()

    def start_s(j):
        b = j & 1
        pltpu.async_copy(rows[b], agg_sh.at[didx(j)], sems[b], add=True)

    def wait_s(j):
        b = j & 1
        pltpu.make_async_copy(rows[b], agg_sh.at[didx(j)], sems[b]).wait()

    load_group(0)
    start_g(0)
    start_g(1)
    wait_g(0)
    start_s(0)
    for p in range(1, CH // 2):
        j0, j1 = 2 * p, 2 * p + 1
        if j0 % GRP == 0:
            load_group(j0 // GRP)
        wait_s(j0 - 2)
        start_g(j0)
        wait_g(j0 - 1)
        start_s(j0 - 1)
        wait_s(j0 - 1)
        start_g(j1)
        wait_g(j0)
        start_s(j0)
    wait_g(CH - 1)
    start_s(CH - 1)
    wait_s(CH - 2)
    wait_s(CH - 1)
    plsc.subcore_barrier()

    def wout(j, carry):
        sl = pl.ds(r0 + j * 128, 128)
        pltpu.sync_copy(agg_sh.at[sl], rows0)
        pltpu.sync_copy(rows0, out_hbm.at[c, sl])
        return carry

    lax.fori_loop(0, RC, wout, 0)


RBLK = 2048
NGRID = N_P // RBLK


def _pre1_body(cnt_ref, f_ref, o_ref):
    cout = cnt_ref[0][:, :1]
    dout = lax.rsqrt(jnp.maximum(cout, 1.0))
    o_ref[...] = f_ref[...] * dout


_CNT_SPEC = pl.BlockSpec((2, RBLK, 128), lambda i: (0, i, 0))
_CNT0_SPEC = pl.BlockSpec((1, RBLK, 128), lambda i: (0, i, 0))
_ROW_SPEC = pl.BlockSpec((RBLK, D), lambda i: (i, 0))
_AGG_SPEC = pl.BlockSpec((2, RBLK, D), lambda i: (0, i, 0))
_W_SPEC = pl.BlockSpec((D, D), lambda i: (0, 0))
_V_SPEC = pl.BlockSpec((1, D), lambda i: (0, 0))

_pre1 = pl.pallas_call(
    _pre1_body,
    grid=(NGRID,),
    in_specs=[_CNT0_SPEC, _ROW_SPEC],
    out_specs=_ROW_SPEC,
    out_shape=jax.ShapeDtypeStruct((N_P, D), jnp.float32),
)


def _make_post(pool):
    def body(x_ref, aggp_ref, cnt_ref, W_ref, Wr_ref, b_ref, br_ref, g_ref, be_ref, *outs):
        i = pl.program_id(0)
        cnt = cnt_ref[...]
        cin = cnt[1][:, :1]
        din = lax.rsqrt(jnp.maximum(cin, 1.0))
        agg = (aggp_ref[0] + aggp_ref[1]) * din
        new = jnp.dot(agg, W_ref[...], preferred_element_type=jnp.float32) + b_ref[...]
        res = jnp.dot(x_ref[...], Wr_ref[...], preferred_element_type=jnp.float32) + br_ref[...]
        res = jnp.maximum(res, 0.0)
        h = (new + res) * BN_S * g_ref[...] + be_ref[...]
        if pool:
            (pool_ref,) = outs
            rows = i * RBLK + lax.broadcasted_iota(jnp.int32, (RBLK, 1), 0)
            ps = jnp.sum(jnp.where(rows < N, h, 0.0), axis=0, keepdims=True)

            @pl.when(i == 0)
            def _():
                pool_ref[...] = ps

            @pl.when(i > 0)
            def _():
                pool_ref[...] += ps
        else:
            h_ref, hs_ref = outs
            h_ref[...] = h
            cout = cnt[0][:, :1]
            dout = lax.rsqrt(jnp.maximum(cout, 1.0))
            hs_ref[...] = h * dout

    return body


_POST_IN_SPECS = [_ROW_SPEC, _AGG_SPEC, _CNT_SPEC,
                  _W_SPEC, _W_SPEC, _V_SPEC, _V_SPEC, _V_SPEC, _V_SPEC]

_post1 = pl.pallas_call(
    _make_post(False),
    grid=(NGRID,),
    in_specs=_POST_IN_SPECS,
    out_specs=[_ROW_SPEC, _ROW_SPEC],
    out_shape=[jax.ShapeDtypeStruct((N_P, D), jnp.float32),
               jax.ShapeDtypeStruct((N_P, D), jnp.float32)],
)

_post2 = pl.pallas_call(
    _make_post(True),
    grid=(NGRID,),
    in_specs=_POST_IN_SPECS,
    out_specs=pl.BlockSpec((1, D), lambda i: (0, 0)),
    out_shape=jax.ShapeDtypeStruct((1, D), jnp.float32),
)


def _head_body(pool_ref, s_ref, Wm1_ref, bm1_ref, gm1_ref, bem1_ref,
               Wm2_ref, bm2_ref, gm2_ref, bem2_ref, Wf_ref, bf_ref, o_ref):
    hg = jnp.maximum(pool_ref[...] * (1.0 / N), 0.0)
    x = jnp.dot(hg, Wm1_ref[0:D, :], preferred_element_type=jnp.float32)
    x = x + jnp.dot(s_ref[...], Wm1_ref[D:D + S_DIM, :], preferred_element_type=jnp.float32)
    x = jnp.maximum(x + bm1_ref[...], 0.0) * BN_S * gm1_ref[...] + bem1_ref[...]
    x = jnp.dot(x, Wm2_ref[...], preferred_element_type=jnp.float32) + bm2_ref[...]
    x = jnp.maximum(x, 0.0) * BN_S * gm2_ref[...] + bem2_ref[...]
    o = jnp.dot(x, Wf_ref[...], preferred_element_type=jnp.float32) + bf_ref[...]
    o_ref[...] = jnp.maximum(o, 0.0)


_head = pl.pallas_call(
    _head_body,
    out_shape=jax.ShapeDtypeStruct((1, NPRED), jnp.float32),
)


def kernel(edge_index, f, ef, s,
           W1, b1, Wr1, br1, g1, be1,
           W2, b2, Wr2, br2, g2, be2,
           Wm1, bm1, gm1, bem1,
           Wm2, bm2, gm2, bem2,
           Wf, bf):
    pad = (N + (jnp.arange(E_P - E, dtype=jnp.int32) % (N_P - N))).astype(jnp.int32)
    src_p = jnp.concatenate([edge_index[0], pad]).reshape(ROWS2D, 128)
    dst_p = jnp.concatenate([edge_index[1], pad]).reshape(ROWS2D, 128)
    f_p = jnp.pad(f, ((0, N_P - N), (0, 0)))

    cnt = _degrees(jnp.concatenate([src_p, dst_p], axis=0))
    h1s = _pre1(cnt, f_p)
    aggp1 = _msgpass(h1s, src_p, dst_p)
    h1, h2s = _post1(f_p, aggp1, cnt,
                     W1, Wr1, b1.reshape(1, D), br1.reshape(1, D),
                     g1.reshape(1, D), be1.reshape(1, D))
    aggp2 = _msgpass(h2s, src_p, dst_p)
    pool = _post2(h1, aggp2, cnt,
                  W2, Wr2, b2.reshape(1, D), br2.reshape(1, D),
                  g2.reshape(1, D), be2.reshape(1, D))
    out = _head(pool, s, Wm1, bm1.reshape(1, MLP_DIM), gm1.reshape(1, MLP_DIM),
                bem1.reshape(1, MLP_DIM), Wm2, bm2.reshape(1, MLP_DIM),
                gm2.reshape(1, MLP_DIM), bem2.reshape(1, MLP_DIM),
                Wf, bf.reshape(1, NPRED))
    return out


# speedup vs baseline: 9.3748x; 1.0063x over previous
import functools
import math

import jax
import jax.numpy as jnp
from jax import lax
from jax.experimental import pallas as pl
from jax.experimental.pallas import tpu as pltpu
from jax.experimental.pallas import tpu_sc as plsc

N = 10000
E = 320000
D = 128
S_DIM = 16
MLP_DIM = 128
NPRED = 1000
EPS = 1e-5
BN_S = 1.0 / math.sqrt(1.0 + EPS)

N_P = 10240
NBLK = N_P // 128
CH = 80
GRP = 16
EPT = CH * 128
E_P = 32 * EPT
ROWS2D = E_P // 128
RPT = N_P // 16
RC = RPT // 128

_MESH = plsc.VectorSubcoreMesh(core_axis_name="c", subcore_axis_name="s")


RPT_D = ROWS2D // 16
GRPD = 16
NGD = RPT_D // GRPD


@functools.partial(
    pl.kernel,
    out_type=jax.ShapeDtypeStruct((2, N_P, 128), jnp.float32),
    mesh=_MESH,
    scratch_types=[
        pltpu.VMEM((GRPD, 128), jnp.int32),
        pltpu.VMEM((GRPD, 128), jnp.int32),
        pltpu.VMEM((128, 128), jnp.float32),
        pltpu.VMEM_SHARED((N_P, 128), jnp.float32),
        pltpu.SemaphoreType.DMA,
        pltpu.SemaphoreType.DMA,
    ],
)
def _degrees(idx_hbm, out_hbm, idxv0, idxv1, ones_v, acc_sh, semd0, semd1):
    c = lax.axis_index("c")
    s = lax.axis_index("s")
    r0 = s * RPT

    def fill0(i, carry):
        for k in range(8):
            ones_v[i, pl.ds(k * 16, 16)] = jnp.zeros((16,), jnp.float32)
        return carry

    lax.fori_loop(0, 128, fill0, 0)

    def zinit(j, carry):
        pltpu.sync_copy(ones_v, acc_sh.at[pl.ds(r0 + j * 128, 128)])
        return carry

    lax.fori_loop(0, RC, zinit, 0)

    def fill1(i, carry):
        for k in range(8):
            ones_v[i, pl.ds(k * 16, 16)] = jnp.float32(1.0) + jnp.zeros((16,), jnp.float32)
        return carry

    lax.fori_loop(0, 128, fill1, 0)
    plsc.subcore_barrier()

    idxv = (idxv0, idxv1)
    semd = (semd0, semd1)

    def load_group(g):
        base = c * ROWS2D + s * RPT_D + g * GRPD
        pltpu.sync_copy(idx_hbm.at[pl.ds(base, GRPD)], idxv[g & 1])

    def fire_group(g):
        for j in range(GRPD):
            pltpu.async_copy(ones_v, acc_sh.at[idxv[g & 1].at[j]], semd[g & 1], add=True)

    def drain_group(g):
        for j in range(GRPD):
            pltpu.make_async_copy(ones_v, acc_sh.at[idxv[g & 1].at[j]], semd[g & 1]).wait()

    load_group(0)
    fire_group(0)
    load_group(1)
    fire_group(1)
    for g in range(2, NGD):
        drain_group(g - 2)
        load_group(g)
        fire_group(g)
    drain_group(NGD - 2)
    drain_group(NGD - 1)
    plsc.subcore_barrier()

    def wout(j, carry):
        sl = pl.ds(r0 + j * 128, 128)
        pltpu.sync_copy(acc_sh.at[sl], ones_v)
        pltpu.sync_copy(ones_v, out_hbm.at[c, sl])
        return carry

    lax.fori_loop(0, RC, wout, 0)


@functools.partial(
    pl.kernel,
    out_type=jax.ShapeDtypeStruct((2, N_P, D), jnp.float32),
    mesh=_MESH,
    scratch_types=[
        pltpu.VMEM((GRP, 128), jnp.int32),
        pltpu.VMEM((GRP, 128), jnp.int32),
        pltpu.VMEM((GRP, 128), jnp.int32),
        pltpu.VMEM((GRP, 128), jnp.int32),
        pltpu.VMEM((128, D), jnp.float32),
        pltpu.VMEM((128, D), jnp.float32),
        pltpu.VMEM_SHARED((N_P, D), jnp.float32),
        pltpu.SemaphoreType.DMA,
        pltpu.SemaphoreType.DMA,
        pltpu.SemaphoreType.DMA,
        pltpu.SemaphoreType.DMA,
    ],
)
def _msgpass(h_hbm, src_hbm, dst_hbm, out_hbm,
             srcv0, srcv1, dstv0, dstv1, rows0, rows1, agg_sh,
             semg0, semg1, sems0, sems1):
    c = lax.axis_index("c")
    s = lax.axis_index("s")
    t = c * 16 + s
    r0 = s * RPT

    def fill(i, carry):
        for k in range(8):
            rows0[i, pl.ds(k * 16, 16)] = jnp.zeros((16,), jnp.float32)
        return carry

    lax.fori_loop(0, 128, fill, 0)

    def zinit(j, carry):
        pltpu.sync_copy(rows0, agg_sh.at[pl.ds(r0 + j * 128, 128)])
        return carry

    lax.fori_loop(0, RC, zinit, 0)
    plsc.subcore_barrier()

    srcv = (srcv0, srcv1)
    dstv = (dstv0, dstv1)
    rows = (rows0, rows1)
    semg = (semg0, semg1)
    sems = (sems0, sems1)

    def load_group(g):
        base = t * CH + g * GRP
        pltpu.sync_copy(src_hbm.at[pl.ds(base, GRP)], srcv[g & 1])
        pltpu.sync_copy(dst_hbm.at[pl.ds(base, GRP)], dstv[g & 1])

    def sidx(j):
        g, r = divmod(j, GRP)
        return srcv[g & 1].at[r]

    def didx(j):
        g, r = divmod(j, GRP)
        return dstv[g & 1].at[r]

    def start_g(j):
        b = j & 1
        pltpu.async_copy(h_hbm.at[sidx(j)], rows[b], semg[b])

    def wait_g(j):
        b = j & 1
        pltpu.make_async_copy(h_hbm.at[sidx(j)], rows[b], semg[b]).wait()

    def start_s(j):
        b = j & 1
        pltpu.async_copy(rows[b], agg_sh.at[didx(j)], sems[b], add=True)

    def wait_s(j):
        b = j & 1
        pltpu.make_async_copy(rows[b], agg_sh.at[didx(j)], sems[b]).wait()

    load_group(0)
    start_g(0)
    start_g(1)
    wait_g(0)
    start_s(0)
    for p in range(1, CH // 2):
        j0, j1 = 2 * p, 2 * p + 1
        if j0 % GRP == 0:
            load_group(j0 // GRP)
        wait_s(j0 - 2)
        start_g(j0)
        wait_g(j0 - 1)
        start_s(j0 - 1)
        wait_s(j0 - 1)
        start_g(j1)
        wait_g(j0)
        start_s(j0)
    wait_g(CH - 1)
    start_s(CH - 1)
    wait_s(CH - 2)
    wait_s(CH - 1)
    plsc.subcore_barrier()

    def wout(j, carry):
        sl = pl.ds(r0 + j * 128, 128)
        pltpu.sync_copy(agg_sh.at[sl], rows0)
        pltpu.sync_copy(rows0, out_hbm.at[c, sl])
        return carry

    lax.fori_loop(0, RC, wout, 0)


RBLK = 2048
NGRID = N_P // RBLK


def _pre1_body(cnt_ref, f_ref, o_ref):
    cout = cnt_ref[0][:, :1]
    dout = lax.rsqrt(jnp.maximum(cout, 1.0))
    o_ref[...] = f_ref[...] * dout


_CNT_SPEC = pl.BlockSpec((2, RBLK, 128), lambda i: (0, i, 0))
_CNT0_SPEC = pl.BlockSpec((1, RBLK, 128), lambda i: (0, i, 0))
_ROW_SPEC = pl.BlockSpec((RBLK, D), lambda i: (i, 0))
_AGG_SPEC = pl.BlockSpec((2, RBLK, D), lambda i: (0, i, 0))
_W_SPEC = pl.BlockSpec((D, D), lambda i: (0, 0))
_V_SPEC = pl.BlockSpec((1, D), lambda i: (0, 0))

_pre1 = pl.pallas_call(
    _pre1_body,
    grid=(NGRID,),
    in_specs=[_CNT0_SPEC, _ROW_SPEC],
    out_specs=_ROW_SPEC,
    out_shape=jax.ShapeDtypeStruct((N_P, D), jnp.float32),
)


def _make_post(pool):
    def body(x_ref, aggp_ref, cnt_ref, W_ref, Wr_ref, b_ref, br_ref, g_ref, be_ref, *rest):
        i = pl.program_id(0)
        cnt = cnt_ref[...]
        cin = cnt[1][:, :1]
        din = lax.rsqrt(jnp.maximum(cin, 1.0))
        agg = (aggp_ref[0] + aggp_ref[1]) * din
        new = jnp.dot(agg, W_ref[...], preferred_element_type=jnp.float32) + b_ref[...]
        res = jnp.dot(x_ref[...], Wr_ref[...], preferred_element_type=jnp.float32) + br_ref[...]
        res = jnp.maximum(res, 0.0)
        h = (new + res) * BN_S * g_ref[...] + be_ref[...]
        if pool:
            (s_ref, Wm1_ref, bm1_ref, gm1_ref, bem1_ref,
             Wm2_ref, bm2_ref, gm2_ref, bem2_ref, Wf_ref, bf_ref,
             o_ref, pool_scr) = rest
            rows = i * RBLK + lax.broadcasted_iota(jnp.int32, (RBLK, 1), 0)
            ps = jnp.sum(jnp.where(rows < N, h, 0.0), axis=0, keepdims=True)

            @pl.when(i == 0)
            def _():
                pool_scr[...] = ps

            @pl.when(i > 0)
            def _():
                pool_scr[...] += ps

            @pl.when(i == NGRID - 1)
            def _():
                hg = jnp.maximum(pool_scr[...] * (1.0 / N), 0.0)
                x1 = jnp.dot(hg, Wm1_ref[0:D, :], preferred_element_type=jnp.float32)
                x1 = x1 + jnp.dot(s_ref[...], Wm1_ref[D:D + S_DIM, :],
                                  preferred_element_type=jnp.float32)
                x1 = jnp.maximum(x1 + bm1_ref[...], 0.0) * BN_S * gm1_ref[...] + bem1_ref[...]
                x1 = jnp.dot(x1, Wm2_ref[...], preferred_element_type=jnp.float32) + bm2_ref[...]
                x1 = jnp.maximum(x1, 0.0) * BN_S * gm2_ref[...] + bem2_ref[...]
                o = jnp.dot(x1, Wf_ref[...], preferred_element_type=jnp.float32) + bf_ref[...]
                o_ref[...] = jnp.maximum(o, 0.0)
        else:
            h_ref, hs_ref = rest
            h_ref[...] = h
            cout = cnt[0][:, :1]
            dout = lax.rsqrt(jnp.maximum(cout, 1.0))
            hs_ref[...] = h * dout

    return body


_POST_IN_SPECS = [_ROW_SPEC, _AGG_SPEC, _CNT_SPEC,
                  _W_SPEC, _W_SPEC, _V_SPEC, _V_SPEC, _V_SPEC, _V_SPEC]

_post1 = pl.pallas_call(
    _make_post(False),
    grid=(NGRID,),
    in_specs=_POST_IN_SPECS,
    out_specs=[_ROW_SPEC, _ROW_SPEC],
    out_shape=[jax.ShapeDtypeStruct((N_P, D), jnp.float32),
               jax.ShapeDtypeStruct((N_P, D), jnp.float32)],
)

_FULL = lambda shape: pl.BlockSpec(shape, lambda i: tuple(0 for _ in shape))

_post2 = pl.pallas_call(
    _make_post(True),
    grid=(NGRID,),
    in_specs=_POST_IN_SPECS + [
        _FULL((1, S_DIM)), _FULL((D + S_DIM, MLP_DIM)), _FULL((1, MLP_DIM)),
        _FULL((1, MLP_DIM)), _FULL((1, MLP_DIM)), _FULL((MLP_DIM, MLP_DIM)),
        _FULL((1, MLP_DIM)), _FULL((1, MLP_DIM)), _FULL((1, MLP_DIM)),
        _FULL((MLP_DIM, NPRED)), _FULL((1, NPRED)),
    ],
    out_specs=pl.BlockSpec((1, NPRED), lambda i: (0, 0)),
    out_shape=jax.ShapeDtypeStruct((1, NPRED), jnp.float32),
    scratch_shapes=[pltpu.VMEM((1, D), jnp.float32)],
)


def kernel(edge_index, f, ef, s,
           W1, b1, Wr1, br1, g1, be1,
           W2, b2, Wr2, br2, g2, be2,
           Wm1, bm1, gm1, bem1,
           Wm2, bm2, gm2, bem2,
           Wf, bf):
    pad = (N + (jnp.arange(E_P - E, dtype=jnp.int32) % (N_P - N))).astype(jnp.int32)
    src_p = jnp.concatenate([edge_index[0], pad]).reshape(ROWS2D, 128)
    dst_p = jnp.concatenate([edge_index[1], pad]).reshape(ROWS2D, 128)
    f_p = jnp.pad(f, ((0, N_P - N), (0, 0)))

    cnt = _degrees(jnp.concatenate([src_p, dst_p], axis=0))
    h1s = _pre1(cnt, f_p)
    aggp1 = _msgpass(h1s, src_p, dst_p)
    h1, h2s = _post1(f_p, aggp1, cnt,
                     W1, Wr1, b1.reshape(1, D), br1.reshape(1, D),
                     g1.reshape(1, D), be1.reshape(1, D))
    aggp2 = _msgpass(h2s, src_p, dst_p)
    out = _post2(h1, aggp2, cnt,
                 W2, Wr2, b2.reshape(1, D), br2.reshape(1, D),
                 g2.reshape(1, D), be2.reshape(1, D),
                 s, Wm1, bm1.reshape(1, MLP_DIM), gm1.reshape(1, MLP_DIM),
                 bem1.reshape(1, MLP_DIM), Wm2, bm2.reshape(1, MLP_DIM),
                 gm2.reshape(1, MLP_DIM), bem2.reshape(1, MLP_DIM),
                 Wf, bf.reshape(1, NPRED))
    return out
